# baseline (speedup 1.0000x reference)
import functools

import jax
import jax.numpy as jnp
from jax import lax
from jax.experimental import pallas as pl
from jax.experimental.pallas import tpu as pltpu
from jax.experimental.pallas import tpu_sc as plsc

B = 16384
D = 16
N_CORES = 2
N_SUBCORES = 16
NW = N_CORES * N_SUBCORES
BPW = B // NW
CHUNK = 128
NCHUNK = BPW // CHUNK
NGROUP = BPW // D


def _body(task_hbm, worker_hbm, tf_hbm, wf_hbm, out_hbm,
          tidx_v, widx_v, tf_v, wf_v, out_v, sem_t, sem_w):
    wid = lax.axis_index("s") * N_CORES + lax.axis_index("c")
    base = wid * BPW

    for j in range(NCHUNK):
        pltpu.sync_copy(task_hbm.at[pl.ds(base + j * CHUNK, CHUNK)],
                        tidx_v.at[j])
        pltpu.sync_copy(worker_hbm.at[pl.ds(base + j * CHUNK, CHUNK)],
                        widx_v.at[j])

    copies = []
    for j in range(NCHUNK):
        copies.append(pltpu.async_copy(
            tf_hbm.at[tidx_v.at[j]], tf_v.at[pl.ds(j * CHUNK, CHUNK)], sem_t))
        copies.append(pltpu.async_copy(
            wf_hbm.at[widx_v.at[j]], wf_v.at[pl.ds(j * CHUNK, CHUNK)], sem_w))
    for c in copies:
        c.wait()

    lane = lax.iota(jnp.int32, D)

    def group(g, carry):
        row_idx = lane + g * D
        acc = jnp.zeros((D,), jnp.float32)
        for k in range(D):
            col_idx = jnp.full((D,), k, jnp.int32)
            a = plsc.load_gather(tf_v, [row_idx, col_idx])
            b = plsc.load_gather(wf_v, [row_idx, col_idx])
            acc = acc + a * b
        score = 1.0 / (1.0 + jnp.exp(-acc))
        out_v[pl.ds(g * D, D)] = score
        return carry

    lax.fori_loop(0, NGROUP, group, 0)

    pltpu.sync_copy(out_v, out_hbm.at[pl.ds(base, BPW)])


@jax.jit
def kernel(task, worker, task_factors, worker_factors):
    mesh = plsc.VectorSubcoreMesh(core_axis_name="c", subcore_axis_name="s")
    run = functools.partial(
        pl.kernel,
        out_type=jax.ShapeDtypeStruct((B,), jnp.float32),
        mesh=mesh,
        scratch_types=[
            pltpu.VMEM((NCHUNK, CHUNK), jnp.int32),
            pltpu.VMEM((NCHUNK, CHUNK), jnp.int32),
            pltpu.VMEM((BPW, D), jnp.float32),
            pltpu.VMEM((BPW, D), jnp.float32),
            pltpu.VMEM((BPW,), jnp.float32),
            pltpu.SemaphoreType.DMA,
            pltpu.SemaphoreType.DMA,
        ],
        compiler_params=pltpu.CompilerParams(
            needs_layout_passes=False, use_tc_tiling_on_sc=False),
    )(_body)
    return run(task, worker, task_factors, worker_factors)
